# baseline (speedup 1.0000x reference)
import jax
import jax.numpy as jnp
from jax import lax
from jax.experimental import pallas as pl
from jax.experimental.pallas import tpu as pltpu

B, L = 64, 25
V, H, D = 30522, 768, 2048
MM, R = 700, 10

_LB = 5
_NL = L // _LB
_MW = MM * R
_CB = 1024
_NC = -(-_MW // _CB)
_N2 = _NL + _NC
_G0 = 3


def _k1_body(ids_ref, table_ref, w0_ref, b0_ref, out_ref, emb_s, sems):
    j = pl.program_id(0)

    def batch(c, start):
        for b in range(B):
            idx = ids_ref[c * B + b]
            cp = pltpu.make_async_copy(
                table_ref.at[pl.ds(idx, 1)],
                emb_s.at[c, pl.ds(b, 1)],
                sems.at[c],
            )
            if start:
                cp.start()
            else:
                cp.wait()

    @pl.when(j == 0)
    def _():
        for c in range(_G0):
            batch(c, True)

    @pl.when(j < L - _G0)
    def _():
        batch(j + _G0, True)

    batch(j, False)

    contrib = lax.dot_general(emb_s[j], w0_ref[...],
                              (((1,), (1,)), ((), ())),
                              preferred_element_type=jnp.float32)

    @pl.when(j == 0)
    def _():
        out_ref[...] = contrib + b0_ref[...]

    @pl.when(j > 0)
    def _():
        out_ref[...] += contrib


def _k2_body(ve_ref, wv_ref, bv_ref, w1_ref, b1_ref, x0_ref,
             wm0_ref, bm0_ref, wm1_ref, bm1_ref,
             wout_ref, bout_ref, lng_ref, lnb_ref, out_ref,
             x1_s, accw_s):
    i = pl.program_id(0)

    @pl.when(i < _NL)
    def _():
        acc = jnp.zeros((B, MM), jnp.float32)
        for k in range(_LB):
            vproj = lax.dot_general(ve_ref[k], wv_ref[...],
                                    (((1,), (1,)), ((), ())),
                                    preferred_element_type=jnp.float32) \
                + bv_ref[...]
            acc += lax.dot_general(
                vproj, w1_ref[:, k * H:(k + 1) * H],
                (((1,), (1,)), ((), ())),
                preferred_element_type=jnp.float32)

        @pl.when(i == 0)
        def _():
            x1_s[...] = acc + b1_ref[...]

        @pl.when(i > 0)
        def _():
            x1_s[...] += acc

    @pl.when(i >= _NL)
    def _():
        c = i - _NL
        m0 = lax.dot_general(x0_ref[...], wm0_ref[...],
                             (((1,), (0,)), ((), ())),
                             preferred_element_type=jnp.float32) \
            + bm0_ref[...]
        m1 = lax.dot_general(x1_s[...], wm1_ref[...],
                             (((1,), (0,)), ((), ())),
                             preferred_element_type=jnp.float32) \
            + bm1_ref[...]
        accw_s[c] = m0 * m1

        @pl.when(i == _N2 - 1)
        def _():
            z = jnp.zeros((B, MM), jnp.float32)
            for r in range(R):
                lo = r * MM
                blk, off = divmod(lo, _CB)
                if off + MM <= _CB:
                    z += accw_s[blk, :, off:off + MM]
                else:
                    cut = _CB - off
                    z += jnp.concatenate(
                        [accw_s[blk, :, off:], accw_s[blk + 1, :, :MM - cut]],
                        axis=1)
            y = lax.dot_general(z, wout_ref[...],
                                (((1,), (0,)), ((), ())),
                                preferred_element_type=jnp.float32) \
                + bout_ref[...]
            mu = jnp.mean(y, axis=-1, keepdims=True)
            var = jnp.mean((y - mu) ** 2, axis=-1, keepdims=True)
            out_ref[...] = (y - mu) * lax.rsqrt(var + 1e-12) * lng_ref[...] \
                + lnb_ref[...]


def kernel(input_ids, token_type_ids, visual_embeds, visual_token_type_ids,
           word_emb, pos_emb, tt_emb, vtt_emb, vpos_emb, Wv, bv,
           W0, b0, W1, b1, Wm0, bm0, Wm1, bm1, Wout, bout, ln_g, ln_b):
    ids = input_ids.T.reshape(-1).astype(jnp.int32)

    x0 = pl.pallas_call(
        _k1_body,
        grid_spec=pltpu.PrefetchScalarGridSpec(
            num_scalar_prefetch=1,
            grid=(L,),
            in_specs=[
                pl.BlockSpec(memory_space=pl.ANY),
                pl.BlockSpec((MM, H), lambda j, ids: (0, j)),
                pl.BlockSpec((1, MM), lambda j, ids: (0, 0)),
            ],
            out_specs=pl.BlockSpec((B, MM), lambda j, ids: (0, 0)),
            scratch_shapes=[
                pltpu.VMEM((L, B, H), jnp.float32),
                pltpu.SemaphoreType.DMA((L,)),
            ],
        ),
        out_shape=jax.ShapeDtypeStruct((B, MM), jnp.float32),
        compiler_params=pltpu.CompilerParams(
            dimension_semantics=("arbitrary",)),
    )(ids, word_emb, W0, b0.reshape(1, MM))

    return pl.pallas_call(
        _k2_body,
        grid=(_N2,),
        in_specs=[
            pl.BlockSpec((_LB, B, D),
                         lambda i: (jnp.minimum(i, _NL - 1), 0, 0)),
            pl.BlockSpec((H, D), lambda i: (0, 0)),
            pl.BlockSpec((1, H), lambda i: (0, 0)),
            pl.BlockSpec((MM, _LB * H),
                         lambda i: (0, jnp.minimum(i, _NL - 1))),
            pl.BlockSpec((1, MM), lambda i: (0, 0)),
            pl.BlockSpec((B, MM), lambda i: (0, 0)),
            pl.BlockSpec((MM, _CB),
                         lambda i: (0, jnp.clip(i - _NL, 0, _NC - 1))),
            pl.BlockSpec((1, _CB),
                         lambda i: (0, jnp.clip(i - _NL, 0, _NC - 1))),
            pl.BlockSpec((MM, _CB),
                         lambda i: (0, jnp.clip(i - _NL, 0, _NC - 1))),
            pl.BlockSpec((1, _CB),
                         lambda i: (0, jnp.clip(i - _NL, 0, _NC - 1))),
            pl.BlockSpec((MM, H), lambda i: (0, 0)),
            pl.BlockSpec((1, H), lambda i: (0, 0)),
            pl.BlockSpec((1, H), lambda i: (0, 0)),
            pl.BlockSpec((1, H), lambda i: (0, 0)),
        ],
        out_specs=pl.BlockSpec((B, H), lambda i: (0, 0)),
        out_shape=jax.ShapeDtypeStruct((B, H), jnp.float32),
        scratch_shapes=[
            pltpu.VMEM((B, MM), jnp.float32),
            pltpu.VMEM((_NC, B, _CB), jnp.float32),
        ],
        compiler_params=pltpu.CompilerParams(
            dimension_semantics=("arbitrary",)),
    )(jnp.transpose(visual_embeds, (1, 0, 2)), Wv, bv.reshape(1, H),
      W1, b1.reshape(1, MM), x0,
      Wm0.T, bm0.reshape(1, _MW), Wm1.T, bm1.reshape(1, _MW),
      Wout.T, bout.reshape(1, H), ln_g.reshape(1, H), ln_b.reshape(1, H))


# speedup vs baseline: 1.0880x; 1.0880x over previous
import jax
import jax.numpy as jnp
from jax import lax
from jax.experimental import pallas as pl
from jax.experimental.pallas import tpu as pltpu

B, L = 64, 25
V, H, D = 30522, 768, 2048
MM, R = 700, 10

_LB = 5
_NL = L // _LB
_MW = MM * R
_CB = 1536
_NC = -(-_MW // _CB)
_N2 = _NL + _NC


def _k1_body(ids_ref, table_ref, w0_ref, b0_ref, out_ref, emb_s, sems):
    j = pl.program_id(0)

    def chunk(c, start):
        for t in range(_LB * B):
            lk, b = divmod(t, B)
            idx = ids_ref[c * _LB * B + t]
            cp = pltpu.make_async_copy(
                table_ref.at[pl.ds(idx, 1)],
                emb_s.at[c * _LB + lk, pl.ds(b, 1)],
                sems.at[c],
            )
            if start:
                cp.start()
            else:
                cp.wait()

    @pl.when(j == 0)
    def _():
        chunk(j, True)

    @pl.when(j < _NL - 1)
    def _():
        chunk(j + 1, True)

    chunk(j, False)

    acc = jnp.zeros((B, MM), jnp.float32)
    for k in range(_LB):
        acc += lax.dot_general(emb_s[j * _LB + k],
                               w0_ref[:, k * H:(k + 1) * H],
                               (((1,), (1,)), ((), ())),
                               preferred_element_type=jnp.float32)

    @pl.when(j == 0)
    def _():
        out_ref[...] = acc + b0_ref[...]

    @pl.when(j > 0)
    def _():
        out_ref[...] += acc


def _k2_body(ve_ref, wv_ref, bv_ref, w1_ref, b1_ref, x0_ref,
             wm0_ref, bm0_ref, wm1_ref, bm1_ref,
             wout_ref, bout_ref, lng_ref, lnb_ref, out_ref,
             x1_s, accw_s):
    i = pl.program_id(0)

    @pl.when(i < _NL)
    def _():
        acc = jnp.zeros((B, MM), jnp.float32)
        for k in range(_LB):
            vproj = lax.dot_general(ve_ref[k], wv_ref[...],
                                    (((1,), (1,)), ((), ())),
                                    preferred_element_type=jnp.float32) \
                + bv_ref[...]
            acc += lax.dot_general(
                vproj, w1_ref[:, k * H:(k + 1) * H],
                (((1,), (1,)), ((), ())),
                preferred_element_type=jnp.float32)

        @pl.when(i == 0)
        def _():
            x1_s[...] = acc + b1_ref[...]

        @pl.when(i > 0)
        def _():
            x1_s[...] += acc

    @pl.when(i >= _NL)
    def _():
        c = i - _NL
        m0 = lax.dot_general(x0_ref[...], wm0_ref[...],
                             (((1,), (0,)), ((), ())),
                             preferred_element_type=jnp.float32) \
            + bm0_ref[...]
        m1 = lax.dot_general(x1_s[...], wm1_ref[...],
                             (((1,), (0,)), ((), ())),
                             preferred_element_type=jnp.float32) \
            + bm1_ref[...]
        accw_s[c] = m0 * m1

        @pl.when(i == _N2 - 1)
        def _():
            z = jnp.zeros((B, MM), jnp.float32)
            for r in range(R):
                lo = r * MM
                blk, off = divmod(lo, _CB)
                if off + MM <= _CB:
                    z += accw_s[blk, :, off:off + MM]
                else:
                    cut = _CB - off
                    z += jnp.concatenate(
                        [accw_s[blk, :, off:], accw_s[blk + 1, :, :MM - cut]],
                        axis=1)
            y = lax.dot_general(z, wout_ref[...],
                                (((1,), (0,)), ((), ())),
                                preferred_element_type=jnp.float32) \
                + bout_ref[...]
            mu = jnp.mean(y, axis=-1, keepdims=True)
            var = jnp.mean((y - mu) ** 2, axis=-1, keepdims=True)
            out_ref[...] = (y - mu) * lax.rsqrt(var + 1e-12) * lng_ref[...] \
                + lnb_ref[...]


def kernel(input_ids, token_type_ids, visual_embeds, visual_token_type_ids,
           word_emb, pos_emb, tt_emb, vtt_emb, vpos_emb, Wv, bv,
           W0, b0, W1, b1, Wm0, bm0, Wm1, bm1, Wout, bout, ln_g, ln_b):
    ids = input_ids.T.reshape(-1).astype(jnp.int32)

    x0 = pl.pallas_call(
        _k1_body,
        grid_spec=pltpu.PrefetchScalarGridSpec(
            num_scalar_prefetch=1,
            grid=(_NL,),
            in_specs=[
                pl.BlockSpec(memory_space=pl.ANY),
                pl.BlockSpec((MM, _LB * H), lambda j, ids: (0, j)),
                pl.BlockSpec((1, MM), lambda j, ids: (0, 0)),
            ],
            out_specs=pl.BlockSpec((B, MM), lambda j, ids: (0, 0)),
            scratch_shapes=[
                pltpu.VMEM((L, B, H), jnp.float32),
                pltpu.SemaphoreType.DMA((_NL,)),
            ],
        ),
        out_shape=jax.ShapeDtypeStruct((B, MM), jnp.float32),
        compiler_params=pltpu.CompilerParams(
            dimension_semantics=("arbitrary",)),
    )(ids, word_emb, W0, b0.reshape(1, MM))

    return pl.pallas_call(
        _k2_body,
        grid=(_N2,),
        in_specs=[
            pl.BlockSpec((_LB, B, D),
                         lambda i: (jnp.minimum(i, _NL - 1), 0, 0)),
            pl.BlockSpec((H, D), lambda i: (0, 0)),
            pl.BlockSpec((1, H), lambda i: (0, 0)),
            pl.BlockSpec((MM, _LB * H),
                         lambda i: (0, jnp.minimum(i, _NL - 1))),
            pl.BlockSpec((1, MM), lambda i: (0, 0)),
            pl.BlockSpec((B, MM), lambda i: (0, 0)),
            pl.BlockSpec((MM, _CB),
                         lambda i: (0, jnp.clip(i - _NL, 0, _NC - 1))),
            pl.BlockSpec((1, _CB),
                         lambda i: (0, jnp.clip(i - _NL, 0, _NC - 1))),
            pl.BlockSpec((MM, _CB),
                         lambda i: (0, jnp.clip(i - _NL, 0, _NC - 1))),
            pl.BlockSpec((1, _CB),
                         lambda i: (0, jnp.clip(i - _NL, 0, _NC - 1))),
            pl.BlockSpec((MM, H), lambda i: (0, 0)),
            pl.BlockSpec((1, H), lambda i: (0, 0)),
            pl.BlockSpec((1, H), lambda i: (0, 0)),
            pl.BlockSpec((1, H), lambda i: (0, 0)),
        ],
        out_specs=pl.BlockSpec((B, H), lambda i: (0, 0)),
        out_shape=jax.ShapeDtypeStruct((B, H), jnp.float32),
        scratch_shapes=[
            pltpu.VMEM((B, MM), jnp.float32),
            pltpu.VMEM((_NC, B, _CB), jnp.float32),
        ],
        compiler_params=pltpu.CompilerParams(
            dimension_semantics=("arbitrary",)),
    )(jnp.transpose(visual_embeds, (1, 0, 2)), Wv, bv.reshape(1, H),
      W1, b1.reshape(1, MM), x0,
      Wm0.T, bm0.reshape(1, _MW), Wm1.T, bm1.reshape(1, _MW),
      Wout.T, bout.reshape(1, H), ln_g.reshape(1, H), ln_b.reshape(1, H))
